# Initial kernel scaffold; baseline (speedup 1.0000x reference)
import functools

import jax
import jax.numpy as jnp
from jax import lax
from jax.experimental import pallas as pl
from jax.experimental.pallas import tpu as pltpu
from jax.experimental.pallas import tpu_sc as plsc

N_NODES = 10000
FEAT = 128
EMB = 64
BATCH = 1024
N_EDGES = 320000

NC = 2
NS = 16
NW = NC * NS

CHUNK = 128
CH_PER_W = -(-N_EDGES // (NW * CHUNK))
EPW = CH_PER_W * CHUNK
EPAD = EPW * NW

ACC_ROWS = 10016
RPS = ACC_ROWS // NS

_THETAS = ((1.0, -1.0, 0.25), (0.0, 1.0, -0.5), (0.0, 0.0, 0.25))

_mesh = plsc.VectorSubcoreMesh(core_axis_name="c", subcore_axis_name="s")


@functools.partial(
    pl.kernel,
    out_type=jax.ShapeDtypeStruct((NC, ACC_ROWS, EMB), jnp.float32),
    mesh=_mesh,
    scratch_types=[
        pltpu.VMEM((CH_PER_W, CHUNK), jnp.int32),
        pltpu.VMEM((CH_PER_W, CHUNK), jnp.int32),
        pltpu.VMEM((CHUNK, EMB), jnp.float32),
        pltpu.VMEM((RPS, EMB), jnp.float32),
        pltpu.VMEM_SHARED((ACC_ROWS, EMB), jnp.float32),
        pltpu.SemaphoreType.DMA,
    ],
)
def _segsum64(xs_hbm, src_hbm, dst_hbm, zeros_hbm, out_hbm,
              sidx_v, didx_v, rows_v, stage_v, acc_sh, sem):
    cid = lax.axis_index("c")
    sid = lax.axis_index("s")
    g = cid * NS + sid
    pltpu.sync_copy(src_hbm.at[g], sidx_v)
    pltpu.sync_copy(dst_hbm.at[g], didx_v)
    pltpu.sync_copy(zeros_hbm.at[pl.ds(sid * RPS, RPS)], stage_v)
    pltpu.sync_copy(stage_v, acc_sh.at[pl.ds(sid * RPS, RPS)])
    plsc.subcore_barrier()

    def body(j, carry):
        pltpu.async_copy(xs_hbm.at[sidx_v.at[j]], rows_v, sem).wait()
        pltpu.sync_copy(rows_v, acc_sh.at[didx_v.at[j]], add=True)
        return carry

    lax.fori_loop(0, CH_PER_W, body, 0)
    plsc.subcore_barrier()
    pltpu.sync_copy(acc_sh.at[pl.ds(sid * RPS, RPS)], stage_v)
    pltpu.sync_copy(stage_v, out_hbm.at[cid, pl.ds(sid * RPS, RPS)])


def _make_gather(D):
    bpw = BATCH // NW

    @functools.partial(
        pl.kernel,
        out_type=jax.ShapeDtypeStruct((BATCH, D), jnp.float32),
        mesh=_mesh,
        scratch_types=[
            pltpu.VMEM((bpw,), jnp.int32),
            pltpu.VMEM((bpw, D), jnp.float32),
            pltpu.SemaphoreType.DMA,
        ],
    )
    def _gather(table_hbm, idx_hbm, out_hbm, idx_v, rows_v, sem):
        cid = lax.axis_index("c")
        sid = lax.axis_index("s")
        base = (cid * NS + sid) * bpw
        pltpu.sync_copy(idx_hbm.at[pl.ds(base, bpw)], idx_v)
        pltpu.async_copy(table_hbm.at[idx_v], rows_v, sem).wait()
        pltpu.sync_copy(rows_v, out_hbm.at[pl.ds(base, bpw)])

    return _gather


_gather_feat = _make_gather(FEAT)
_gather_emb = _make_gather(EMB)


def _mlp_body(f_ref, w1_ref, b1_ref, w2_ref, b2_ref, deg2_ref,
              h_ref, y_ref, d_ref):
    f = f_ref[...]
    h = jnp.maximum(jnp.dot(f, w1_ref[...], preferred_element_type=jnp.float32)
                    + b1_ref[...][None, :], 0.0)
    h = jnp.maximum(jnp.dot(h, w2_ref[...], preferred_element_type=jnp.float32)
                    + b2_ref[...][None, :], 0.0)
    deg = deg2_ref[0, :N_NODES, 0] + deg2_ref[1, :N_NODES, 0]
    d = lax.rsqrt(jnp.maximum(deg, 1.0))
    h_ref[...] = h
    y_ref[...] = d[:, None] * h
    d_ref[...] = d[:, None]


def _mlp(features, W1, b1, W2, b2, deg2):
    return pl.pallas_call(
        _mlp_body,
        out_shape=(
            jax.ShapeDtypeStruct((N_NODES, EMB), jnp.float32),
            jax.ShapeDtypeStruct((N_NODES, EMB), jnp.float32),
            jax.ShapeDtypeStruct((N_NODES, 1), jnp.float32),
        ),
    )(features, W1, b1, W2, b2, deg2)


def _combine_body(a, b, c, h_ref, t_ref, p2_ref, d_ref, out_ref, y_ref):
    p = p2_ref[0, :N_NODES, :] + p2_ref[1, :N_NODES, :]
    d = d_ref[...]
    t = t_ref[...]
    out = c * (t - d * p)
    if b != 0.0:
        out = out + b * t
    if a != 0.0:
        out = out + a * h_ref[...]
    out_ref[...] = out
    y_ref[...] = d * out


def _combine(a, b, c, h, t, p2, d):
    body = functools.partial(_combine_body, a, b, c)
    return pl.pallas_call(
        body,
        out_shape=(
            jax.ShapeDtypeStruct((N_NODES, EMB), jnp.float32),
            jax.ShapeDtypeStruct((N_NODES, EMB), jnp.float32),
        ),
    )(h, t, p2, d)


def _final_body(fsel_ref, hsel_ref, w3_ref, b3_ref, w_ref, wclf_ref, bclf_ref,
                comb_ref, cs_ref):
    fsel = fsel_ref[...]
    spe = jnp.dot(hsel_ref[...], w3_ref[...],
                  preferred_element_type=jnp.float32) + b3_ref[...][None, :]
    center_h = jnp.dot(fsel, w_ref[...], preferred_element_type=jnp.float32)
    agg = jnp.dot(spe, w_ref[...], preferred_element_type=jnp.float32)
    comb_ref[...] = jnp.maximum(center_h + agg, 0.0)
    cs_ref[...] = jnp.dot(fsel, wclf_ref[...],
                          preferred_element_type=jnp.float32) + bclf_ref[...][None, :]


def _final(fsel, hsel, W3, b3, weight, W_clf, b_clf):
    return pl.pallas_call(
        _final_body,
        out_shape=(
            jax.ShapeDtypeStruct((BATCH, EMB), jnp.float32),
            jax.ShapeDtypeStruct((BATCH, 2), jnp.float32),
        ),
    )(fsel, hsel, W3, b3, weight, W_clf, b_clf)


def kernel(nodes, labels, edge_index, features, W_clf, b_clf,
           W1, b1, W2, b2, W3, b3, weight):
    src = edge_index[0]
    dst = edge_index[1]
    pad = EPAD - N_EDGES
    src_p = jnp.concatenate(
        [src, jnp.zeros((pad,), jnp.int32)]).reshape(NW, CH_PER_W, CHUNK)
    dst_p = jnp.concatenate(
        [dst, jnp.full((pad,), N_NODES, jnp.int32)]).reshape(NW, CH_PER_W, CHUNK)
    zeros64 = jnp.zeros((ACC_ROWS, EMB), jnp.float32)
    ones_table = jnp.ones((N_NODES, EMB), jnp.float32)

    deg2 = _segsum64(ones_table, src_p, dst_p, zeros64)

    h, y, d = _mlp(features, W1, b1, W2, b2, deg2)

    for t0, t1, t2 in _THETAS:
        p2 = _segsum64(y, src_p, dst_p, zeros64)
        tmp1, y = _combine(0.0, 0.0, 1.0, h, h, p2, d)
        p2 = _segsum64(y, src_p, dst_p, zeros64)
        h, y = _combine(t0, t1, t2, h, tmp1, p2, d)

    fsel = _gather_feat(features, nodes)
    hsel = _gather_emb(h, nodes)
    combined, center_scores = _final(fsel, hsel, W3, b3, weight, W_clf, b_clf)
    return (combined, center_scores)


# speedup vs baseline: 7.2625x; 7.2625x over previous
import functools

import jax
import jax.numpy as jnp
from jax import lax
from jax.experimental import pallas as pl
from jax.experimental.pallas import tpu as pltpu
from jax.experimental.pallas import tpu_sc as plsc

N_NODES = 10000
FEAT = 128
EMB = 64
BATCH = 1024
N_EDGES = 320000

NC = 2
NS = 16
NW = NC * NS

CHUNK = 128
CH_PER_W = -(-N_EDGES // (NW * CHUNK))
EPW = CH_PER_W * CHUNK
EPAD = EPW * NW

ACC_ROWS = 10112
RPS = ACC_ROWS // NS

_THETAS = ((1.0, -1.0, 0.25), (0.0, 1.0, -0.5), (0.0, 0.0, 0.25))

_mesh = plsc.VectorSubcoreMesh(core_axis_name="c", subcore_axis_name="s")


@functools.partial(
    pl.kernel,
    out_type=jax.ShapeDtypeStruct((NC, ACC_ROWS, EMB), jnp.float32),
    mesh=_mesh,
    scratch_types=[
        pltpu.VMEM((CH_PER_W, CHUNK), jnp.int32),
        pltpu.VMEM((CH_PER_W, CHUNK), jnp.int32),
        pltpu.VMEM((CHUNK, EMB), jnp.float32),
        pltpu.VMEM((RPS, EMB), jnp.float32),
        pltpu.VMEM_SHARED((ACC_ROWS, EMB), jnp.float32),
        pltpu.SemaphoreType.DMA,
    ],
    compiler_params=pltpu.CompilerParams(use_tc_tiling_on_sc=False),
)
def _segsum64(xs_hbm, src_hbm, dst_hbm, zeros_hbm, out_hbm,
              sidx_v, didx_v, rows_v, stage_v, acc_sh, sem):
    cid = lax.axis_index("c")
    sid = lax.axis_index("s")
    g = cid * NS + sid
    pltpu.sync_copy(src_hbm.at[g], sidx_v)
    pltpu.sync_copy(dst_hbm.at[g], didx_v)
    pltpu.sync_copy(zeros_hbm.at[pl.ds(sid * RPS, RPS)], stage_v)
    pltpu.sync_copy(stage_v, acc_sh.at[pl.ds(sid * RPS, RPS)])
    plsc.subcore_barrier()

    def body(j, carry):
        pltpu.async_copy(xs_hbm.at[sidx_v.at[j]], rows_v, sem).wait()
        pltpu.sync_copy(rows_v, acc_sh.at[didx_v.at[j]], add=True)
        return carry

    lax.fori_loop(0, CH_PER_W, body, 0)
    plsc.subcore_barrier()
    pltpu.sync_copy(acc_sh.at[pl.ds(sid * RPS, RPS)], stage_v)
    pltpu.sync_copy(stage_v, out_hbm.at[cid, pl.ds(sid * RPS, RPS)])


def _make_gather(D):
    bpw = BATCH // NW

    @functools.partial(
        pl.kernel,
        out_type=jax.ShapeDtypeStruct((BATCH, D), jnp.float32),
        mesh=_mesh,
        scratch_types=[
            pltpu.VMEM((bpw,), jnp.int32),
            pltpu.VMEM((bpw, D), jnp.float32),
            pltpu.SemaphoreType.DMA,
        ],
        compiler_params=pltpu.CompilerParams(use_tc_tiling_on_sc=False),
    )
    def _gather(table_hbm, idx_hbm, out_hbm, idx_v, rows_v, sem):
        cid = lax.axis_index("c")
        sid = lax.axis_index("s")
        base = (cid * NS + sid) * bpw
        pltpu.sync_copy(idx_hbm.at[pl.ds(base, bpw)], idx_v)
        pltpu.async_copy(table_hbm.at[idx_v], rows_v, sem).wait()
        pltpu.sync_copy(rows_v, out_hbm.at[pl.ds(base, bpw)])

    return _gather


_gather_feat = _make_gather(FEAT)
_gather_emb = _make_gather(EMB)


def _mlp_body(f_ref, w1_ref, b1_ref, w2_ref, b2_ref, deg2_ref,
              h_ref, y_ref, d_ref):
    f = f_ref[...]
    h = jnp.maximum(jnp.dot(f, w1_ref[...], preferred_element_type=jnp.float32)
                    + b1_ref[...][None, :], 0.0)
    h = jnp.maximum(jnp.dot(h, w2_ref[...], preferred_element_type=jnp.float32)
                    + b2_ref[...][None, :], 0.0)
    deg = deg2_ref[0, :N_NODES, 0] + deg2_ref[1, :N_NODES, 0]
    d = lax.rsqrt(jnp.maximum(deg, 1.0))
    h_ref[...] = h
    y_ref[...] = d[:, None] * h
    d_ref[...] = d[:, None]


def _mlp(features, W1, b1, W2, b2, deg2):
    return pl.pallas_call(
        _mlp_body,
        out_shape=(
            jax.ShapeDtypeStruct((N_NODES, EMB), jnp.float32),
            jax.ShapeDtypeStruct((N_NODES, EMB), jnp.float32),
            jax.ShapeDtypeStruct((N_NODES, 1), jnp.float32),
        ),
    )(features, W1, b1, W2, b2, deg2)


def _combine_body(a, b, c, h_ref, t_ref, p2_ref, d_ref, out_ref, y_ref):
    p = p2_ref[0, :N_NODES, :] + p2_ref[1, :N_NODES, :]
    d = d_ref[...]
    t = t_ref[...]
    out = c * (t - d * p)
    if b != 0.0:
        out = out + b * t
    if a != 0.0:
        out = out + a * h_ref[...]
    out_ref[...] = out
    y_ref[...] = d * out


def _combine(a, b, c, h, t, p2, d):
    body = functools.partial(_combine_body, a, b, c)
    return pl.pallas_call(
        body,
        out_shape=(
            jax.ShapeDtypeStruct((N_NODES, EMB), jnp.float32),
            jax.ShapeDtypeStruct((N_NODES, EMB), jnp.float32),
        ),
    )(h, t, p2, d)


def _final_body(fsel_ref, hsel_ref, w3_ref, b3_ref, w_ref, wclf_ref, bclf_ref,
                comb_ref, cs_ref):
    fsel = fsel_ref[...]
    spe = jnp.dot(hsel_ref[...], w3_ref[...],
                  preferred_element_type=jnp.float32) + b3_ref[...][None, :]
    center_h = jnp.dot(fsel, w_ref[...], preferred_element_type=jnp.float32)
    agg = jnp.dot(spe, w_ref[...], preferred_element_type=jnp.float32)
    comb_ref[...] = jnp.maximum(center_h + agg, 0.0)
    cs_ref[...] = jnp.dot(fsel, wclf_ref[...],
                          preferred_element_type=jnp.float32) + bclf_ref[...][None, :]


def _final(fsel, hsel, W3, b3, weight, W_clf, b_clf):
    return pl.pallas_call(
        _final_body,
        out_shape=(
            jax.ShapeDtypeStruct((BATCH, EMB), jnp.float32),
            jax.ShapeDtypeStruct((BATCH, 2), jnp.float32),
        ),
    )(fsel, hsel, W3, b3, weight, W_clf, b_clf)


def kernel(nodes, labels, edge_index, features, W_clf, b_clf,
           W1, b1, W2, b2, W3, b3, weight):
    src = edge_index[0]
    dst = edge_index[1]
    pad = EPAD - N_EDGES
    src_p = jnp.concatenate(
        [src, jnp.zeros((pad,), jnp.int32)]).reshape(NW, CH_PER_W, CHUNK)
    dst_p = jnp.concatenate(
        [dst, jnp.full((pad,), N_NODES, jnp.int32)]).reshape(NW, CH_PER_W, CHUNK)
    zeros64 = jnp.zeros((ACC_ROWS, EMB), jnp.float32)
    ones_table = jnp.ones((N_NODES, EMB), jnp.float32)

    deg2 = _segsum64(ones_table, src_p, dst_p, zeros64)

    h, y, d = _mlp(features, W1, b1, W2, b2, deg2)

    for t0, t1, t2 in _THETAS:
        p2 = _segsum64(y, src_p, dst_p, zeros64)
        tmp1, y = _combine(0.0, 0.0, 1.0, h, h, p2, d)
        p2 = _segsum64(y, src_p, dst_p, zeros64)
        h, y = _combine(t0, t1, t2, h, tmp1, p2, d)

    fsel = _gather_feat(features, nodes)
    hsel = _gather_emb(h, nodes)
    combined, center_scores = _final(fsel, hsel, W3, b3, weight, W_clf, b_clf)
    return (combined, center_scores)
